# baseline (speedup 1.0000x reference)
import functools

import jax
import jax.numpy as jnp
from jax import lax
from jax.experimental import pallas as pl
from jax.experimental.pallas import tpu as pltpu
from jax.experimental.pallas import tpu_sc as plsc

N = 10000
E = 320000
C = 128
D = 16
OUT = 64

NC = 2
NS = 16
NW = NC * NS
NPAD = 10240
RPT = NPAD // NS
EPW = E // NW
K = 8
NBLK = EPW // K
NCH = 10
CHB = NBLK // NCH

LC = (0.99988891, -0.49770282, 0.31687717, -0.19223705, 0.0841971, -0.01787732)
EC = (1.0000000755142242, 0.6931472067162231, 0.24022107265229542,
      0.055503272022581436, 0.009676041929919721, 0.0013400436945785181)
RMAGIC = 0x7EF311C3


def _fexp(v):
    y = jnp.clip(v * 1.4426950408889634, -126.0, 126.0)
    n = (y + 12582912.0) - 12582912.0
    f = y - n
    p = ((((EC[5] * f + EC[4]) * f + EC[3]) * f + EC[2]) * f + EC[1]) * f + EC[0]
    ni = lax.convert_element_type(n, jnp.int32)
    s = lax.bitcast_convert_type(lax.shift_left(ni + 127, 23), jnp.float32)
    return p * s


def _frcp(d):
    r = lax.bitcast_convert_type(
        RMAGIC - lax.bitcast_convert_type(d, jnp.int32), jnp.float32
    )
    r = r * (2.0 - d * r)
    r = r * (2.0 - d * r)
    return r


def _proj_body(x_ref, w_ref, a_ref, b_ref):
    h = jnp.dot(x_ref[...], w_ref[...], preferred_element_type=jnp.float32)
    a_ref[...] = h[:, : 2 * C]
    b_ref[...] = h[:, 2 * C :]


def _proj(x, w):
    bn = 1000
    return pl.pallas_call(
        _proj_body,
        grid=(N // bn,),
        in_specs=[
            pl.BlockSpec((bn, C), lambda i: (i, 0)),
            pl.BlockSpec((C, 4 * C), lambda i: (0, 0)),
        ],
        out_specs=[
            pl.BlockSpec((bn, 2 * C), lambda i: (i, 0)),
            pl.BlockSpec((bn, 2 * C), lambda i: (i, 0)),
        ],
        out_shape=[
            jax.ShapeDtypeStruct((N, 2 * C), jnp.float32),
            jax.ShapeDtypeStruct((N, 2 * C), jnp.float32),
        ],
    )(x, w)


def _ew_body(e_ref, w_ref, o_ref):
    o_ref[...] = jnp.dot(e_ref[...], w_ref[...], preferred_element_type=jnp.float32)


def _ew(edge_w_aug, we_aug):
    be = 3200
    return pl.pallas_call(
        _ew_body,
        grid=(E // be,),
        in_specs=[
            pl.BlockSpec((be, 24), lambda i: (i, 0)),
            pl.BlockSpec((24, 2 * C), lambda i: (0, 0)),
        ],
        out_specs=pl.BlockSpec((be, 2 * C), lambda i: (i, 0)),
        out_shape=jax.ShapeDtypeStruct((E, 2 * C), jnp.float32),
    )(edge_w_aug, we_aug)


def _combine_body(x_ref, g_ref, o_ref, *, relu):
    h = x_ref[...] + g_ref[0] + g_ref[1]
    if relu:
        h = jnp.maximum(h, 0.0)
    o_ref[...] = h


def _combine(x, agg2, relu):
    bn = 1000
    return pl.pallas_call(
        functools.partial(_combine_body, relu=relu),
        grid=(N // bn,),
        in_specs=[
            pl.BlockSpec((bn, C), lambda i: (i, 0)),
            pl.BlockSpec((NC, bn, C), lambda i: (0, i, 0)),
        ],
        out_specs=pl.BlockSpec((bn, C), lambda i: (i, 0)),
        out_shape=jax.ShapeDtypeStruct((N, C), jnp.float32),
    )(x, agg2)


def _pool_body(h_ref, wl_ref, bl_ref, o_ref, acc_ref):
    i = pl.program_id(0)

    @pl.when(i == 0)
    def _():
        acc_ref[...] = jnp.zeros_like(acc_ref)

    hb = h_ref[...].reshape(-1, 8, C)
    acc_ref[...] += jnp.sum(hb, axis=0)

    @pl.when(i == pl.num_programs(0) - 1)
    def _():
        tot = jnp.sum(acc_ref[...], axis=0, keepdims=True) * (1.0 / N)
        o_ref[...] = (
            jnp.dot(tot, wl_ref[...], preferred_element_type=jnp.float32)
            + bl_ref[...]
        )


def _pool(h, wlin_t, blin):
    bn = 1000
    return pl.pallas_call(
        _pool_body,
        grid=(N // bn,),
        in_specs=[
            pl.BlockSpec((bn, C), lambda i: (i, 0)),
            pl.BlockSpec((C, OUT), lambda i: (0, 0)),
            pl.BlockSpec((1, OUT), lambda i: (0, 0)),
        ],
        out_specs=pl.BlockSpec((1, OUT), lambda i: (0, 0)),
        out_shape=jax.ShapeDtypeStruct((1, OUT), jnp.float32),
        scratch_shapes=[pltpu.VMEM((8, C), jnp.float32)],
    )(h, wlin_t, blin)


def _edge_body(
    dst_hbm, src_hbm, a_tab, b_tab, ew_hbm, zero_hbm, out_hbm,
    idx_d, idx_s, a0, b0, e0, a1, b1, e1, m_rows, agg_sh,
    sa0, sb0, se0, sa1, sb1, se1,
):
    cid = lax.axis_index("c")
    sid = lax.axis_index("s")
    wid = sid * NC + cid

    pltpu.sync_copy(
        zero_hbm.at[pl.ds(sid * RPT, RPT)], agg_sh.at[pl.ds(sid * RPT, RPT)]
    )
    plsc.subcore_barrier()

    def compute(ar, br, er):
        @plsc.parallel_loop(0, K, step=1, unroll=4)
        def _(k):
            for c in range(C // 16):
                slf = pl.ds(c * 16, 16)
                sls = pl.ds(C + c * 16, 16)
                sf = ar[k, slf] + br[k, slf] + er[k, slf]
                ss = ar[k, sls] + br[k, sls] + er[k, sls]
                gate = _frcp(1.0 + _fexp(-sf))
                u = _fexp(-jnp.abs(ss))
                p = ((((LC[5] * u + LC[4]) * u + LC[3]) * u + LC[2]) * u + LC[1]) * u
                p = (p + LC[0]) * u
                sp = jnp.maximum(ss, 0.0) + p
                m_rows[k, slf] = gate * sp

    def chunk_body(ch, carry):
        pltpu.sync_copy(dst_hbm.at[wid, ch], idx_d)
        pltpu.sync_copy(src_hbm.at[wid, ch], idx_s)

        def issue(b, ar, br, er, sa, sb, se):
            pltpu.async_copy(a_tab.at[idx_d.at[b]], ar, sa)
            pltpu.async_copy(b_tab.at[idx_s.at[b]], br, sb)
            pltpu.async_copy(ew_hbm.at[wid, ch, b], er, se)

        def wait(b, ar, br, er, sa, sb, se):
            pltpu.make_async_copy(a_tab.at[idx_d.at[b]], ar, sa).wait()
            pltpu.make_async_copy(b_tab.at[idx_s.at[b]], br, sb).wait()
            pltpu.make_async_copy(ew_hbm.at[wid, ch, b], er, se).wait()

        def do_block(b, ar, br, er, sa, sb, se):
            wait(b, ar, br, er, sa, sb, se)
            compute(ar, br, er)
            pltpu.sync_copy(m_rows, agg_sh.at[idx_d.at[b]], add=True)

        issue(0, a0, b0, e0, sa0, sb0, se0)

        def pair_body(j, carry2):
            bb = 2 * j
            issue(bb + 1, a1, b1, e1, sa1, sb1, se1)
            do_block(bb, a0, b0, e0, sa0, sb0, se0)
            issue(bb + 2, a0, b0, e0, sa0, sb0, se0)
            do_block(bb + 1, a1, b1, e1, sa1, sb1, se1)
            return carry2

        lax.fori_loop(0, (CHB - 1) // 2, pair_body, 0)
        do_block(CHB - 1, a0, b0, e0, sa0, sb0, se0)
        return carry

    lax.fori_loop(0, NCH, chunk_body, 0)

    plsc.subcore_barrier()
    pltpu.sync_copy(
        agg_sh.at[pl.ds(sid * RPT, RPT)],
        out_hbm.at[cid, pl.ds(sid * RPT, RPT)],
    )


def _edge_stage(dst3, src3, a_tab, b_tab, ew4, zeros_n):
    mesh = plsc.VectorSubcoreMesh(core_axis_name="c", subcore_axis_name="s")
    call = pl.kernel(
        _edge_body,
        out_type=jax.ShapeDtypeStruct((NC, NPAD, C), jnp.float32),
        mesh=mesh,
        scratch_types=[
            pltpu.VMEM((CHB, K), jnp.int32),
            pltpu.VMEM((CHB, K), jnp.int32),
            pltpu.VMEM((K, 2 * C), jnp.float32),
            pltpu.VMEM((K, 2 * C), jnp.float32),
            pltpu.VMEM((K, 2 * C), jnp.float32),
            pltpu.VMEM((K, 2 * C), jnp.float32),
            pltpu.VMEM((K, 2 * C), jnp.float32),
            pltpu.VMEM((K, 2 * C), jnp.float32),
            pltpu.VMEM((K, C), jnp.float32),
            pltpu.VMEM_SHARED((NPAD, C), jnp.float32),
            pltpu.SemaphoreType.DMA,
            pltpu.SemaphoreType.DMA,
            pltpu.SemaphoreType.DMA,
            pltpu.SemaphoreType.DMA,
            pltpu.SemaphoreType.DMA,
            pltpu.SemaphoreType.DMA,
        ],
    )
    return call(dst3, src3, a_tab, b_tab, ew4, zeros_n)


def _layer_weights(Wf, bf, Ws, bs):
    w_node = jnp.concatenate(
        [Wf[:, :C].T, Ws[:, :C].T, Wf[:, C : 2 * C].T, Ws[:, C : 2 * C].T], axis=1
    )
    we = jnp.concatenate([Wf[:, 2 * C :].T, Ws[:, 2 * C :].T], axis=1)
    bias = jnp.concatenate([bf, bs])[None, :]
    we_aug = jnp.concatenate(
        [we, bias, jnp.zeros((24 - D - 1, 2 * C), jnp.float32)], axis=0
    )
    return w_node, we_aug


@jax.jit
def _run(x, edge_index, edge_w, weights):
    x = x.astype(jnp.float32)
    src3 = edge_index[0].reshape(NW, NCH, CHB, K)
    dst3 = edge_index[1].reshape(NW, NCH, CHB, K)
    edge_w_aug = jnp.concatenate(
        [
            edge_w.astype(jnp.float32),
            jnp.ones((E, 1), jnp.float32),
            jnp.zeros((E, 24 - D - 1), jnp.float32),
        ],
        axis=1,
    )
    zeros_n = jnp.zeros((NPAD, C), jnp.float32)

    h = x
    for li, (Wf, bf, Ws, bs) in enumerate(weights[:3]):
        w_node, we_aug = _layer_weights(Wf, bf, Ws, bs)
        a_tab, b_tab = _proj(h, w_node)
        ew4 = _ew(edge_w_aug, we_aug).reshape(NW, NCH, CHB, K, 2 * C)
        agg2 = _edge_stage(dst3, src3, a_tab, b_tab, ew4, zeros_n)
        h = _combine(h, agg2, relu=(li < 2))

    wlin_t, blin = weights[3]
    return _pool(h, wlin_t, blin[None, :])


def kernel(x, edge_index, edge_w, Wf1, bf1, Ws1, bs1, Wf2, bf2, Ws2, bs2,
           Wf3, bf3, Ws3, bs3, Wlin, blin):
    weights = (
        (Wf1, bf1, Ws1, bs1),
        (Wf2, bf2, Ws2, bs2),
        (Wf3, bf3, Ws3, bs3),
        (Wlin.T, blin),
    )
    return _run(x, edge_index, edge_w, weights)


# speedup vs baseline: 3.2748x; 3.2748x over previous
import functools

import jax
import jax.numpy as jnp
from jax import lax
from jax.experimental import pallas as pl
from jax.experimental.pallas import tpu as pltpu
from jax.experimental.pallas import tpu_sc as plsc

N = 10000
E = 320000
C = 128
D = 16
OUT = 64

NC = 2
NS = 16
NW = NC * NS
NPAD = 10240
RPT = NPAD // NS
EPW = E // NW

KA = 40
NBA = EPW // KA
NCA = 5
CBA = NBA // NCA

KB = 40
NBB = EPW // KB
NCB = 5
CBB = NBB // NCB


def _proj_body(x_ref, w_ref, a_ref, b_ref):
    h = jnp.dot(x_ref[...], w_ref[...], preferred_element_type=jnp.float32)
    a_ref[...] = h[:, : 2 * C]
    b_ref[...] = h[:, 2 * C :]


def _proj(x, w):
    bn = 1000
    return pl.pallas_call(
        _proj_body,
        grid=(N // bn,),
        in_specs=[
            pl.BlockSpec((bn, C), lambda i: (i, 0)),
            pl.BlockSpec((C, 4 * C), lambda i: (0, 0)),
        ],
        out_specs=[
            pl.BlockSpec((bn, 2 * C), lambda i: (i, 0)),
            pl.BlockSpec((bn, 2 * C), lambda i: (i, 0)),
        ],
        out_shape=[
            jax.ShapeDtypeStruct((N, 2 * C), jnp.float32),
            jax.ShapeDtypeStruct((N, 2 * C), jnp.float32),
        ],
    )(x, w)


def _gates_body(s_ref, e_ref, w_ref, m_ref):
    ewp = jnp.dot(e_ref[...], w_ref[...], preferred_element_type=jnp.float32)
    z = s_ref[...] + ewp
    f = z[:, :C]
    s = z[:, C:]
    gate = 1.0 / (1.0 + jnp.exp(-f))
    sp = jnp.maximum(s, 0.0) + jnp.log(1.0 + jnp.exp(-jnp.abs(s)))
    m_ref[...] = gate * sp


def _gates(sums, edge_w_aug, we_aug):
    be = 2000
    return pl.pallas_call(
        _gates_body,
        grid=(E // be,),
        in_specs=[
            pl.BlockSpec((be, 2 * C), lambda i: (i, 0)),
            pl.BlockSpec((be, 24), lambda i: (i, 0)),
            pl.BlockSpec((24, 2 * C), lambda i: (0, 0)),
        ],
        out_specs=pl.BlockSpec((be, C), lambda i: (i, 0)),
        out_shape=jax.ShapeDtypeStruct((E, C), jnp.float32),
    )(sums, edge_w_aug, we_aug)


def _combine_body(x_ref, g_ref, o_ref, *, relu):
    h = x_ref[...] + g_ref[0] + g_ref[1]
    if relu:
        h = jnp.maximum(h, 0.0)
    o_ref[...] = h


def _combine(x, agg2, relu):
    bn = 1000
    return pl.pallas_call(
        functools.partial(_combine_body, relu=relu),
        grid=(N // bn,),
        in_specs=[
            pl.BlockSpec((bn, C), lambda i: (i, 0)),
            pl.BlockSpec((NC, bn, C), lambda i: (0, i, 0)),
        ],
        out_specs=pl.BlockSpec((bn, C), lambda i: (i, 0)),
        out_shape=jax.ShapeDtypeStruct((N, C), jnp.float32),
    )(x, agg2)


def _pool_body(h_ref, wl_ref, bl_ref, o_ref, acc_ref):
    i = pl.program_id(0)

    @pl.when(i == 0)
    def _():
        acc_ref[...] = jnp.zeros_like(acc_ref)

    hb = h_ref[...].reshape(-1, 8, C)
    acc_ref[...] += jnp.sum(hb, axis=0)

    @pl.when(i == pl.num_programs(0) - 1)
    def _():
        tot = jnp.sum(acc_ref[...], axis=0, keepdims=True) * (1.0 / N)
        o_ref[...] = (
            jnp.dot(tot, wl_ref[...], preferred_element_type=jnp.float32)
            + bl_ref[...]
        )


def _pool(h, wlin_t, blin):
    bn = 1000
    return pl.pallas_call(
        _pool_body,
        grid=(N // bn,),
        in_specs=[
            pl.BlockSpec((bn, C), lambda i: (i, 0)),
            pl.BlockSpec((C, OUT), lambda i: (0, 0)),
            pl.BlockSpec((1, OUT), lambda i: (0, 0)),
        ],
        out_specs=pl.BlockSpec((1, OUT), lambda i: (0, 0)),
        out_shape=jax.ShapeDtypeStruct((1, OUT), jnp.float32),
        scratch_shapes=[pltpu.VMEM((8, C), jnp.float32)],
    )(h, wlin_t, blin)


def _gather_body(
    comb_hbm, tab, out_hbm,
    idx_c, g0, g1, s0, s1,
    sg0, sg1, so0, so1,
):
    cid = lax.axis_index("c")
    sid = lax.axis_index("s")
    wid = sid * NC + cid

    def compute(gr, sr):
        @plsc.parallel_loop(0, KA, step=1, unroll=4)
        def _(k):
            for c in range(2 * C // 16):
                sl = pl.ds(c * 16, 16)
                sr[k, sl] = gr[k, sl] + gr[KA + k, sl]

    def chunk_body(ch, carry):
        pltpu.sync_copy(comb_hbm.at[wid, ch], idx_c)

        def issue(b, gr, sg):
            pltpu.async_copy(tab.at[idx_c.at[b]], gr, sg)

        def wait_g(b, gr, sg):
            pltpu.make_async_copy(tab.at[idx_c.at[b]], gr, sg).wait()

        def wait_s(b, sr, so):
            pltpu.make_async_copy(sr, out_hbm.at[wid, ch, b], so).wait()

        def do_block(b, gr, sg, sr, so, swait):
            wait_g(b, gr, sg)
            if swait is not None:
                swait()
            compute(gr, sr)
            pltpu.async_copy(sr, out_hbm.at[wid, ch, b], so)

        issue(0, g0, sg0)

        def pair_body(j, carry2):
            bb = 2 * j
            issue(bb + 1, g1, sg1)

            def w0():
                @pl.when(j > 0)
                def _():
                    wait_s(bb, s0, so0)

            do_block(bb, g0, sg0, s0, so0, w0)
            issue(bb + 2, g0, sg0)

            def w1():
                @pl.when(j > 0)
                def _():
                    wait_s(bb + 1, s1, so1)

            do_block(bb + 1, g1, sg1, s1, so1, w1)
            return carry2

        lax.fori_loop(0, CBA // 2 - 1, pair_body, 0)
        bb = CBA - 2
        issue(bb + 1, g1, sg1)
        do_block(bb, g0, sg0, s0, so0, lambda: wait_s(bb, s0, so0))
        do_block(bb + 1, g1, sg1, s1, so1, lambda: wait_s(bb + 1, s1, so1))
        wait_s(bb, s0, so0)
        wait_s(bb + 1, s1, so1)
        return carry

    lax.fori_loop(0, NCA, chunk_body, 0)


def _gather_stage(comb4, tab):
    mesh = plsc.VectorSubcoreMesh(core_axis_name="c", subcore_axis_name="s")
    call = pl.kernel(
        _gather_body,
        out_type=jax.ShapeDtypeStruct((NW, NCA, CBA, KA, 2 * C), jnp.float32),
        mesh=mesh,
        scratch_types=[
            pltpu.VMEM((CBA, 2 * KA), jnp.int32),
            pltpu.VMEM((2 * KA, 2 * C), jnp.float32),
            pltpu.VMEM((2 * KA, 2 * C), jnp.float32),
            pltpu.VMEM((KA, 2 * C), jnp.float32),
            pltpu.VMEM((KA, 2 * C), jnp.float32),
            pltpu.SemaphoreType.DMA,
            pltpu.SemaphoreType.DMA,
            pltpu.SemaphoreType.DMA,
            pltpu.SemaphoreType.DMA,
        ],
    )
    return call(comb4, tab)


def _scatter_body(
    dst_hbm, m_hbm, zero_hbm, out_hbm,
    idx_d, m0, m1, agg_sh,
    sm0, sm1,
):
    cid = lax.axis_index("c")
    sid = lax.axis_index("s")
    wid = sid * NC + cid

    pltpu.sync_copy(
        zero_hbm.at[pl.ds(sid * RPT, RPT)], agg_sh.at[pl.ds(sid * RPT, RPT)]
    )
    plsc.subcore_barrier()

    def chunk_body(ch, carry):
        pltpu.sync_copy(dst_hbm.at[wid, ch], idx_d)

        def issue(b, mr, sm):
            pltpu.async_copy(m_hbm.at[wid, ch, b], mr, sm)

        def wait_m(b, mr, sm):
            pltpu.make_async_copy(m_hbm.at[wid, ch, b], mr, sm).wait()

        def do_block(b, mr, sm):
            wait_m(b, mr, sm)
            pltpu.sync_copy(mr, agg_sh.at[idx_d.at[b]], add=True)

        issue(0, m0, sm0)

        def pair_body(j, carry2):
            bb = 2 * j
            issue(bb + 1, m1, sm1)
            do_block(bb, m0, sm0)
            issue(bb + 2, m0, sm0)
            do_block(bb + 1, m1, sm1)
            return carry2

        lax.fori_loop(0, CBB // 2 - 1, pair_body, 0)
        bb = CBB - 2
        issue(bb + 1, m1, sm1)
        do_block(bb, m0, sm0)
        do_block(bb + 1, m1, sm1)
        return carry

    lax.fori_loop(0, NCB, chunk_body, 0)

    plsc.subcore_barrier()
    pltpu.sync_copy(
        agg_sh.at[pl.ds(sid * RPT, RPT)],
        out_hbm.at[cid, pl.ds(sid * RPT, RPT)],
    )


def _scatter_stage(dstb, m5, zeros_n):
    mesh = plsc.VectorSubcoreMesh(core_axis_name="c", subcore_axis_name="s")
    call = pl.kernel(
        _scatter_body,
        out_type=jax.ShapeDtypeStruct((NC, NPAD, C), jnp.float32),
        mesh=mesh,
        scratch_types=[
            pltpu.VMEM((CBB, KB), jnp.int32),
            pltpu.VMEM((KB, C), jnp.float32),
            pltpu.VMEM((KB, C), jnp.float32),
            pltpu.VMEM_SHARED((NPAD, C), jnp.float32),
            pltpu.SemaphoreType.DMA,
            pltpu.SemaphoreType.DMA,
        ],
    )
    return call(dstb, m5, zeros_n)


def _layer_weights(Wf, bf, Ws, bs):
    w_node = jnp.concatenate(
        [Wf[:, :C].T, Ws[:, :C].T, Wf[:, C : 2 * C].T, Ws[:, C : 2 * C].T], axis=1
    )
    we = jnp.concatenate([Wf[:, 2 * C :].T, Ws[:, 2 * C :].T], axis=1)
    bias = jnp.concatenate([bf, bs])[None, :]
    we_aug = jnp.concatenate(
        [we, bias, jnp.zeros((24 - D - 1, 2 * C), jnp.float32)], axis=0
    )
    return w_node, we_aug


@jax.jit
def _run(x, edge_index, edge_w, weights):
    x = x.astype(jnp.float32)
    src4 = edge_index[0].reshape(NW, NCA, CBA, KA)
    dst4 = edge_index[1].reshape(NW, NCA, CBA, KA)
    comb4 = jnp.concatenate([dst4, src4 + N], axis=-1)
    dstb = edge_index[1].reshape(NW, NCB, CBB, KB)
    edge_w_aug = jnp.concatenate(
        [
            edge_w.astype(jnp.float32),
            jnp.ones((E, 1), jnp.float32),
            jnp.zeros((E, 24 - D - 1), jnp.float32),
        ],
        axis=1,
    )
    zeros_n = jnp.zeros((NPAD, C), jnp.float32)

    h = x
    for li, (Wf, bf, Ws, bs) in enumerate(weights[:3]):
        w_node, we_aug = _layer_weights(Wf, bf, Ws, bs)
        a_tab, b_tab = _proj(h, w_node)
        tab = jnp.concatenate([a_tab, b_tab], axis=0)
        sums = _gather_stage(comb4, tab).reshape(E, 2 * C)
        m = _gates(sums, edge_w_aug, we_aug)
        m5 = m.reshape(NW, NCB, CBB, KB, C)
        agg2 = _scatter_stage(dstb, m5, zeros_n)
        h = _combine(h, agg2, relu=(li < 2))

    wlin_t, blin = weights[3]
    return _pool(h, wlin_t, blin[None, :])


def kernel(x, edge_index, edge_w, Wf1, bf1, Ws1, bs1, Wf2, bf2, Ws2, bs2,
           Wf3, bf3, Ws3, bs3, Wlin, blin):
    weights = (
        (Wf1, bf1, Ws1, bs1),
        (Wf2, bf2, Ws2, bs2),
        (Wf3, bf3, Ws3, bs3),
        (Wlin.T, blin),
    )
    return _run(x, edge_index, edge_w, weights)
